# baseline (speedup 1.0000x reference)
import numpy as np

import jax
import jax.numpy as jnp
from jax import lax
from jax.experimental import pallas as pl
from jax.experimental.pallas import tpu as pltpu
from jax.experimental.pallas import tpu_sc as plsc

N = 10000
E = 320000
D = 128

NC = 2
NS = 16
NW = NC * NS
HD = D // NC
HW = HD // 2
K = 128
KAS = 128
CHD = 80
CHAS = 160
E_PAD = NW * CHD * K
N_ACC = 10016
N_ACCD = 10112

_f32 = jnp.float32


def _mesh():
    return plsc.VectorSubcoreMesh(
        core_axis_name="c", subcore_axis_name="s", num_cores=NC, num_subcores=NS
    )


def _perm64():
    p = []
    for g2 in range(2):
        p += [32 * g2 + 2 * w for w in range(16)]
        p += [32 * g2 + 2 * w + 1 for w in range(16)]
    return np.array(p, dtype=np.int32)


_P64 = _perm64()
_P128 = np.concatenate([_P64, _P64 + HD])
_INV128 = np.argsort(_P128).astype(np.int32)


def _deg_body(col_hbm, z_hbm, out_hbm, cidx, ones_v, sdeg):
    c = lax.axis_index("c")
    s = lax.axis_index("s")
    wid = c * NS + s
    pltpu.sync_copy(col_hbm.at[wid], cidx)
    for g in range(K // 16):
        ones_v[pl.ds(g * 16, 16)] = jnp.ones((16,), _f32)

    @pl.when(s == 0)
    def _zero():
        pltpu.sync_copy(z_hbm, sdeg)

    plsc.subcore_barrier()

    def body(ch, carry):
        pltpu.sync_copy(ones_v, sdeg.at[cidx.at[ch]], add=True)
        return carry

    lax.fori_loop(0, CHD, body, 0)
    plsc.subcore_barrier()

    @pl.when(s == 0)
    def _writeback():
        pltpu.sync_copy(sdeg, out_hbm.at[c])


_deg_call = pl.kernel(
    _deg_body,
    out_type=jax.ShapeDtypeStruct((NC, N_ACCD), _f32),
    mesh=_mesh(),
    scratch_types=[
        pltpu.VMEM((CHD, K), jnp.int32),
        pltpu.VMEM((K,), _f32),
        pltpu.VMEM_SHARED((N_ACCD,), _f32),
    ],
)


def _agg_body(h_hbm, row_hbm, col_hbm, z_hbm, out_hbm,
              ridx, cidx, gbb, gbf, tbl, acc,
              g0, g1, s0, s1):
    c = lax.axis_index("c")
    s = lax.axis_index("s")
    gsem = [g0, g1]
    ssem = [s0, s1]

    za = pl.multiple_of(s * 624, 8)
    pltpu.sync_copy(z_hbm.at[pl.ds(0, 624)], acc.at[pl.ds(za, 624)])

    @pl.when(s == NS - 1)
    def _ztail():
        pltpu.sync_copy(z_hbm.at[pl.ds(0, 32)], acc.at[pl.ds(9984, 32)])

    ts = pl.multiple_of(s * 624, 8)
    pltpu.sync_copy(h_hbm.at[c].at[pl.ds(ts, 624)], tbl.at[pl.ds(ts, 624)])

    @pl.when(s == NS - 1)
    def _ttail():
        pltpu.sync_copy(h_hbm.at[c].at[pl.ds(9984, 16)],
                        tbl.at[pl.ds(9984, 16)])

    pltpu.sync_copy(row_hbm.at[s], ridx)
    pltpu.sync_copy(col_hbm.at[s], cidx)
    plsc.subcore_barrier()

    def fire_gather(ch, b):
        pltpu.async_copy(tbl.at[ridx.at[ch]], gbb.at[b], gsem[b])

    def wait_gather(b):
        pltpu.make_async_copy(tbl.at[ridx.at[0]], gbb.at[b], gsem[b]).wait()

    def fire_scatter(ch, b):
        pltpu.async_copy(gbf.at[b], acc.at[cidx.at[ch]], ssem[b], add=True)

    def wait_scatter(b):
        pltpu.make_async_copy(gbf.at[b], acc.at[cidx.at[0]], ssem[b]).wait()

    def convert(b):
        mask = jnp.full((16,), -65536, jnp.int32)
        for r in range(KAS):
            for g2 in range(2):
                w = gbb[b, r, pl.ds(16 * g2, 16)]
                gbf[b, r, pl.ds(32 * g2, 16)] = plsc.bitcast(
                    lax.shift_left(w, 16), _f32)
                gbf[b, r, pl.ds(32 * g2 + 16, 16)] = plsc.bitcast(
                    jnp.bitwise_and(w, mask), _f32)

    fire_gather(0, 0)
    for ch in (0, 1):
        b = ch & 1
        wait_gather(b)
        fire_gather(ch + 1, 1 - b)
        convert(b)
        fire_scatter(ch, b)

    def body(o, carry):
        for b in (0, 1):
            ch = 2 * o + b
            wait_gather(b)
            fire_gather(ch + 1, 1 - b)
            wait_scatter(b)
            convert(b)
            fire_scatter(ch, b)
        return carry

    lax.fori_loop(1, CHAS // 2 - 1, body, 0)

    for ch in (CHAS - 2, CHAS - 1):
        b = ch & 1
        wait_gather(b)
        if ch + 1 < CHAS:
            fire_gather(ch + 1, 1 - b)
        wait_scatter(b)
        convert(b)
        fire_scatter(ch, b)
    for b in (0, 1):
        wait_scatter(b)

    plsc.subcore_barrier()
    wb = pl.multiple_of(s * 624, 8)
    pltpu.sync_copy(acc.at[pl.ds(wb, 624)], out_hbm.at[c].at[pl.ds(wb, 624)])

    @pl.when(s == NS - 1)
    def _tail():
        pltpu.sync_copy(acc.at[pl.ds(9984, 16)],
                        out_hbm.at[c].at[pl.ds(9984, 16)])


_agg_call = pl.kernel(
    _agg_body,
    out_type=jax.ShapeDtypeStruct((NC, N, HD), _f32),
    mesh=_mesh(),
    compiler_params=pltpu.CompilerParams(
        use_tc_tiling_on_sc=False, needs_layout_passes=False),
    scratch_types=[
        pltpu.VMEM((CHAS, KAS), jnp.int32),
        pltpu.VMEM((CHAS, KAS), jnp.int32),
        pltpu.VMEM((2, KAS, HW), jnp.int32),
        pltpu.VMEM((2, KAS, HD), _f32),
        pltpu.VMEM_SHARED((N, HW), jnp.int32),
        pltpu.VMEM_SHARED((N_ACC, HD), _f32),
        pltpu.SemaphoreType.DMA,
        pltpu.SemaphoreType.DMA,
        pltpu.SemaphoreType.DMA,
        pltpu.SemaphoreType.DMA,
    ],
)


BN = 1000


def _dinv(deg_ref):
    d = deg_ref[0] + deg_ref[1]
    return jnp.where(d > 0, lax.rsqrt(d), 0.0)


def _split_store_bf16(o_ref, h):
    hb = h.astype(jnp.bfloat16)
    o_ref[0] = hb[:, :HD]
    o_ref[1] = hb[:, HD:]


def _mat1_body(x_ref, wt_ref, deg_ref, o_ref):
    h = jnp.dot(x_ref[...], wt_ref[...], preferred_element_type=_f32)
    _split_store_bf16(o_ref, h * _dinv(deg_ref))


def _post1_body(agg_ref, deg_ref, b_ref, wt_ref, h1_ref, h2p_ref):
    dinv = _dinv(deg_ref)
    a = jnp.concatenate([agg_ref[0], agg_ref[1]], axis=1)
    h1 = jnp.maximum(a * dinv + b_ref[...], 0.0)
    h1_ref[...] = h1
    h2p = jnp.dot(h1, wt_ref[...], preferred_element_type=_f32) * dinv
    _split_store_bf16(h2p_ref, h2p)


def _post2_body(agg_ref, deg_ref, b_ref, h2_ref):
    dinv = _dinv(deg_ref)
    a = jnp.concatenate([agg_ref[0], agg_ref[1]], axis=1)
    h2_ref[...] = jnp.maximum(a * dinv + b_ref[...], 0.0)


_split_spec = pl.BlockSpec((NC, BN, HD), lambda i: (0, i, 0))
_deg_spec = pl.BlockSpec((NC, BN, 1), lambda i: (0, i, 0))
_row_spec = pl.BlockSpec((BN, D), lambda i: (i, 0))
_w_spec = pl.BlockSpec((D, D), lambda i: (0, 0))
_b_spec = pl.BlockSpec((1, D), lambda i: (0, 0))
_splitb_shape = jax.ShapeDtypeStruct((NC, N, HD), jnp.bfloat16)
_row_shape = jax.ShapeDtypeStruct((N, D), _f32)


def _mat1(x, wt, degc):
    return pl.pallas_call(
        _mat1_body,
        grid=(N // BN,),
        in_specs=[_row_spec, _w_spec, _deg_spec],
        out_specs=_split_spec,
        out_shape=_splitb_shape,
    )(x, wt, degc)


def _post1(aggs, degc, brow, wt):
    return pl.pallas_call(
        _post1_body,
        grid=(N // BN,),
        in_specs=[_split_spec, _deg_spec, _b_spec, _w_spec],
        out_specs=[_row_spec, _split_spec],
        out_shape=[_row_shape, _splitb_shape],
    )(aggs, degc, brow, wt)


def _post2(aggs, degc, brow):
    return pl.pallas_call(
        _post2_body,
        grid=(N // BN,),
        in_specs=[_split_spec, _deg_spec, _b_spec],
        out_specs=_row_spec,
        out_shape=_row_shape,
    )(aggs, degc, brow)


def kernel(x, edge_index, edge_features, W1, b1, W2, b2):
    del edge_features
    row = edge_index[0].astype(jnp.int32)
    col = edge_index[1].astype(jnp.int32)
    pad = E_PAD - E
    rowp = jnp.concatenate([row, jnp.zeros((pad,), jnp.int32)])
    colp = jnp.concatenate(
        [col, N + (jnp.arange(pad, dtype=jnp.int32) % (N_ACC - N))]
    )
    rowa = rowp.reshape(NS, CHAS, KAS)
    cola = colp.reshape(NS, CHAS, KAS)
    cold = colp.reshape(NW, CHD, K)
    zrows = jnp.zeros((624, HD), _f32)
    zdeg = jnp.zeros((N_ACCD,), _f32)

    degp = _deg_call(cold, zdeg)
    degc = degp[:, :N].reshape(NC, N, 1)

    b1p = b1[_P128].reshape(1, D)
    b2p = b2[_P128].reshape(1, D)
    w2tp = W2.T[_P128, :]

    h1t = _mat1(x, W1.T, degc)
    h1w = lax.bitcast_convert_type(
        h1t.reshape(NC, N, HW, 2), jnp.int32)
    aggs1 = _agg_call(h1w, rowa, cola, zrows)
    h1p, h2t = _post1(aggs1, degc, b1p, w2tp)
    h2w = lax.bitcast_convert_type(h2t.reshape(NC, N, HW, 2), jnp.int32)
    aggs2 = _agg_call(h2w, rowa, cola, zrows)
    h2p = _post2(aggs2, degc, b2p)
    h1 = h1p[:, _INV128]
    h2 = h2p[:, _INV128]
    return (h1, h2)


# speedup vs baseline: 1.1550x; 1.1550x over previous
import numpy as np

import jax
import jax.numpy as jnp
from jax import lax
from jax.experimental import pallas as pl
from jax.experimental.pallas import tpu as pltpu
from jax.experimental.pallas import tpu_sc as plsc

N = 10000
E = 320000
D = 128

NC = 2
NS = 16
NW = NC * NS
HD = D // NC
HW = HD // 2
K = 128
KAS = 32
CHD = 80
CHAS = 640
E_PAD = NW * CHD * K
N_ACC = 10016
N_ACCD = 10112

_f32 = jnp.float32


def _mesh():
    return plsc.VectorSubcoreMesh(
        core_axis_name="c", subcore_axis_name="s", num_cores=NC, num_subcores=NS
    )


def _perm64():
    p = []
    for g2 in range(2):
        p += [32 * g2 + 2 * w for w in range(16)]
        p += [32 * g2 + 2 * w + 1 for w in range(16)]
    return np.array(p, dtype=np.int32)


_P64 = _perm64()
_P128 = np.concatenate([_P64, _P64 + HD])
_INV128 = np.argsort(_P128).astype(np.int32)


def _deg_body(col_hbm, z_hbm, out_hbm, cidx, ones_v, sdeg):
    c = lax.axis_index("c")
    s = lax.axis_index("s")
    wid = c * NS + s
    pltpu.sync_copy(col_hbm.at[wid], cidx)
    for g in range(K // 16):
        ones_v[pl.ds(g * 16, 16)] = jnp.ones((16,), _f32)

    @pl.when(s == 0)
    def _zero():
        pltpu.sync_copy(z_hbm, sdeg)

    plsc.subcore_barrier()

    def body(ch, carry):
        pltpu.sync_copy(ones_v, sdeg.at[cidx.at[ch]], add=True)
        return carry

    lax.fori_loop(0, CHD, body, 0)
    plsc.subcore_barrier()

    @pl.when(s == 0)
    def _writeback():
        pltpu.sync_copy(sdeg, out_hbm.at[c])


_deg_call = pl.kernel(
    _deg_body,
    out_type=jax.ShapeDtypeStruct((NC, N_ACCD), _f32),
    mesh=_mesh(),
    scratch_types=[
        pltpu.VMEM((CHD, K), jnp.int32),
        pltpu.VMEM((K,), _f32),
        pltpu.VMEM_SHARED((N_ACCD,), _f32),
    ],
)


def _agg_body(h_hbm, row_hbm, col_hbm, z_hbm, out_hbm,
              ridx, cidx, gbb, gbf, tbl, acc,
              g0, g1, s0, s1):
    c = lax.axis_index("c")
    s = lax.axis_index("s")
    gsem = [g0, g1]
    ssem = [s0, s1]

    za = pl.multiple_of(s * 624, 8)
    pltpu.sync_copy(z_hbm.at[pl.ds(0, 624)], acc.at[pl.ds(za, 624)])

    @pl.when(s == NS - 1)
    def _ztail():
        pltpu.sync_copy(z_hbm.at[pl.ds(0, 32)], acc.at[pl.ds(9984, 32)])

    ts = pl.multiple_of(s * 624, 8)
    pltpu.sync_copy(h_hbm.at[c].at[pl.ds(ts, 624)], tbl.at[pl.ds(ts, 624)])

    @pl.when(s == NS - 1)
    def _ttail():
        pltpu.sync_copy(h_hbm.at[c].at[pl.ds(9984, 16)],
                        tbl.at[pl.ds(9984, 16)])

    pltpu.sync_copy(row_hbm.at[s], ridx)
    pltpu.sync_copy(col_hbm.at[s], cidx)
    plsc.subcore_barrier()

    def fire_gather(ch, b):
        pltpu.async_copy(tbl.at[ridx.at[ch]], gbb.at[b], gsem[b])

    def wait_gather(b):
        pltpu.make_async_copy(tbl.at[ridx.at[0]], gbb.at[b], gsem[b]).wait()

    def fire_scatter(ch, b):
        pltpu.async_copy(gbf.at[b], acc.at[cidx.at[ch]], ssem[b], add=True)

    def wait_scatter(b):
        pltpu.make_async_copy(gbf.at[b], acc.at[cidx.at[0]], ssem[b]).wait()

    def convert(b):
        mask = jnp.full((16,), -65536, jnp.int32)
        for r in range(KAS):
            for g2 in range(2):
                w = gbb[b, r, pl.ds(16 * g2, 16)]
                gbf[b, r, pl.ds(32 * g2, 16)] = plsc.bitcast(
                    lax.shift_left(w, 16), _f32)
                gbf[b, r, pl.ds(32 * g2 + 16, 16)] = plsc.bitcast(
                    jnp.bitwise_and(w, mask), _f32)

    fire_gather(0, 0)
    for ch in (0, 1):
        b = ch & 1
        wait_gather(b)
        fire_gather(ch + 1, 1 - b)
        convert(b)
        fire_scatter(ch, b)

    def body(o, carry):
        for b in (0, 1):
            ch = 2 * o + b
            wait_gather(b)
            fire_gather(ch + 1, 1 - b)
            wait_scatter(b)
            convert(b)
            fire_scatter(ch, b)
        return carry

    lax.fori_loop(1, CHAS // 2 - 1, body, 0)

    for ch in (CHAS - 2, CHAS - 1):
        b = ch & 1
        wait_gather(b)
        if ch + 1 < CHAS:
            fire_gather(ch + 1, 1 - b)
        wait_scatter(b)
        convert(b)
        fire_scatter(ch, b)
    for b in (0, 1):
        wait_scatter(b)

    plsc.subcore_barrier()
    wb = pl.multiple_of(s * 624, 8)
    pltpu.sync_copy(acc.at[pl.ds(wb, 624)], out_hbm.at[c].at[pl.ds(wb, 624)])

    @pl.when(s == NS - 1)
    def _tail():
        pltpu.sync_copy(acc.at[pl.ds(9984, 16)],
                        out_hbm.at[c].at[pl.ds(9984, 16)])


_agg_call = pl.kernel(
    _agg_body,
    out_type=jax.ShapeDtypeStruct((NC, N, HD), _f32),
    mesh=_mesh(),
    compiler_params=pltpu.CompilerParams(
        use_tc_tiling_on_sc=False, needs_layout_passes=False),
    scratch_types=[
        pltpu.VMEM((CHAS, KAS), jnp.int32),
        pltpu.VMEM((CHAS, KAS), jnp.int32),
        pltpu.VMEM((2, KAS, HW), jnp.int32),
        pltpu.VMEM((2, KAS, HD), _f32),
        pltpu.VMEM_SHARED((N, HW), jnp.int32),
        pltpu.VMEM_SHARED((N_ACC, HD), _f32),
        pltpu.SemaphoreType.DMA,
        pltpu.SemaphoreType.DMA,
        pltpu.SemaphoreType.DMA,
        pltpu.SemaphoreType.DMA,
    ],
)


BN = 1000


def _dinv(deg_ref):
    d = deg_ref[0] + deg_ref[1]
    return jnp.where(d > 0, lax.rsqrt(d), 0.0)


def _split_store_bf16(o_ref, h):
    hb = h.astype(jnp.bfloat16)
    o_ref[0] = hb[:, :HD]
    o_ref[1] = hb[:, HD:]


def _mat1_body(x_ref, wt_ref, deg_ref, o_ref):
    h = jnp.dot(x_ref[...], wt_ref[...], preferred_element_type=_f32)
    _split_store_bf16(o_ref, h * _dinv(deg_ref))


def _post1_body(agg_ref, deg_ref, b_ref, wt_ref, h1_ref, h2p_ref):
    dinv = _dinv(deg_ref)
    a = jnp.concatenate([agg_ref[0], agg_ref[1]], axis=1)
    h1 = jnp.maximum(a * dinv + b_ref[...], 0.0)
    h1_ref[...] = h1
    h2p = jnp.dot(h1, wt_ref[...], preferred_element_type=_f32) * dinv
    _split_store_bf16(h2p_ref, h2p)


def _post2_body(agg_ref, deg_ref, b_ref, h2_ref):
    dinv = _dinv(deg_ref)
    a = jnp.concatenate([agg_ref[0], agg_ref[1]], axis=1)
    h2_ref[...] = jnp.maximum(a * dinv + b_ref[...], 0.0)


_split_spec = pl.BlockSpec((NC, BN, HD), lambda i: (0, i, 0))
_deg_spec = pl.BlockSpec((NC, BN, 1), lambda i: (0, i, 0))
_row_spec = pl.BlockSpec((BN, D), lambda i: (i, 0))
_w_spec = pl.BlockSpec((D, D), lambda i: (0, 0))
_b_spec = pl.BlockSpec((1, D), lambda i: (0, 0))
_splitb_shape = jax.ShapeDtypeStruct((NC, N, HD), jnp.bfloat16)
_row_shape = jax.ShapeDtypeStruct((N, D), _f32)


def _mat1(x, wt, degc):
    return pl.pallas_call(
        _mat1_body,
        grid=(N // BN,),
        in_specs=[_row_spec, _w_spec, _deg_spec],
        out_specs=_split_spec,
        out_shape=_splitb_shape,
    )(x, wt, degc)


def _post1(aggs, degc, brow, wt):
    return pl.pallas_call(
        _post1_body,
        grid=(N // BN,),
        in_specs=[_split_spec, _deg_spec, _b_spec, _w_spec],
        out_specs=[_row_spec, _split_spec],
        out_shape=[_row_shape, _splitb_shape],
    )(aggs, degc, brow, wt)


def _post2(aggs, degc, brow):
    return pl.pallas_call(
        _post2_body,
        grid=(N // BN,),
        in_specs=[_split_spec, _deg_spec, _b_spec],
        out_specs=_row_spec,
        out_shape=_row_shape,
    )(aggs, degc, brow)


def kernel(x, edge_index, edge_features, W1, b1, W2, b2):
    del edge_features
    row = edge_index[0].astype(jnp.int32)
    col = edge_index[1].astype(jnp.int32)
    pad = E_PAD - E
    rowp = jnp.concatenate([row, jnp.zeros((pad,), jnp.int32)])
    colp = jnp.concatenate(
        [col, N + (jnp.arange(pad, dtype=jnp.int32) % (N_ACC - N))]
    )
    rowa = rowp.reshape(NS, CHAS, KAS)
    cola = colp.reshape(NS, CHAS, KAS)
    cold = colp.reshape(NW, CHD, K)
    zrows = jnp.zeros((624, HD), _f32)
    zdeg = jnp.zeros((N_ACCD,), _f32)

    degp = _deg_call(cold, zdeg)
    degc = degp[:, :N].reshape(NC, N, 1)

    b1p = b1[_P128].reshape(1, D)
    b2p = b2[_P128].reshape(1, D)
    w2tp = W2.T[_P128, :]

    h1t = _mat1(x, W1.T, degc)
    h1w = lax.bitcast_convert_type(
        h1t.reshape(NC, N, HW, 2), jnp.int32)
    aggs1 = _agg_call(h1w, rowa, cola, zrows)
    h1p, h2t = _post1(aggs1, degc, b1p, w2tp)
    h2w = lax.bitcast_convert_type(h2t.reshape(NC, N, HW, 2), jnp.int32)
    aggs2 = _agg_call(h2w, rowa, cola, zrows)
    h2p = _post2(aggs2, degc, b2p)
    h1 = h1p[:, _INV128]
    h2 = h2p[:, _INV128]
    return (h1, h2)
